# baseline (speedup 1.0000x reference)
import functools

import jax
import jax.numpy as jnp
from jax import lax
from jax.experimental import pallas as pl
from jax.experimental.pallas import tpu as pltpu
from jax.experimental.pallas import tpu_sc as plsc

L = 16
D = 8
NC = 2
NS = 16
NW = NC * NS
BR = 1000


def _build_sc_cols(n):
    groups = n // L
    gbase = groups // NW
    extra = groups - gbase * NW
    wcut = NW - extra
    gmax = gbase + (1 if extra else 0)
    rows_w = gbase * L
    rows_max = gmax * L

    g1 = groups // NS
    rem1 = groups - g1 * NS
    ch1 = g1 * L
    tail_off = ch1 * NS

    mesh = plsc.VectorSubcoreMesh(core_axis_name="c", subcore_axis_name="s")

    @functools.partial(
        pl.kernel,
        out_type=jax.ShapeDtypeStruct((n,), jnp.int32),
        mesh=mesh,
        scratch_types=[
            pltpu.VMEM((ch1,), jnp.int32),
            pltpu.VMEM((L,), jnp.int32),
            pltpu.VMEM((rows_max,), jnp.int32),
            pltpu.VMEM((rows_max,), jnp.int32),
            pltpu.VMEM((L,), jnp.int32),
            pltpu.VMEM((NS * L,), jnp.int32),
            pltpu.VMEM((L,), jnp.int32),
            pltpu.VMEM_SHARED((NS * L,), jnp.int32),
        ],
        compiler_params=pltpu.CompilerParams(needs_layout_passes=False),
    )
    def call(ls_hbm, lim_hbm, cols_hbm,
             vals1_v, tail_v, vals2_v, cols_v,
             minvec_v, allmins_v, lim_v, mins_sh):
        c = lax.axis_index("c")
        s = lax.axis_index("s")
        w = s * NC + c

        pltpu.sync_copy(ls_hbm.at[pl.ds(s * ch1, ch1)], vals1_v)
        if rem1:
            pltpu.sync_copy(
                ls_hbm.at[pl.ds(tail_off + lax.rem(s, rem1) * L, L)], tail_v)
            m0 = tail_v[...]
        else:
            m0 = jnp.full((L,), jnp.iinfo(jnp.int32).max, jnp.int32)

        def mstep(i, m):
            return jnp.minimum(m, vals1_v[pl.ds(i * L, L)])
        m = lax.fori_loop(0, g1, mstep, m0)
        minvec_v[...] = m
        pltpu.sync_copy(minvec_v, mins_sh.at[pl.ds(s * L, L)])
        plsc.subcore_barrier()
        pltpu.sync_copy(mins_sh, allmins_v)

        def mstep2(i, m):
            return jnp.minimum(m, allmins_v[pl.ds(i * L, L)])
        mall = lax.fori_loop(0, NS, mstep2,
                             jnp.full((L,), jnp.iinfo(jnp.int32).max,
                                      jnp.int32))
        gmin = mall[0]
        for j in range(1, L):
            gmin = jnp.minimum(gmin, mall[j])

        base = rows_w * w + L * jnp.maximum(w - wcut, 0)
        ng = gbase + jnp.where(w >= wcut, 1, 0) if extra else gbase
        pltpu.sync_copy(ls_hbm.at[pl.ds(base, rows_max)], vals2_v)
        pltpu.sync_copy(lim_hbm, lim_v)
        lim = lim_v[...]

        def wstep(i, carry):
            v = vals2_v[pl.ds(i * L, L)] - gmin
            v = jnp.minimum(v, lim)
            v = jnp.maximum(v, 0)
            cols_v[pl.ds(i * L, L)] = v
            return carry
        lax.fori_loop(0, ng, wstep, 0)

        pltpu.sync_copy(cols_v.at[pl.ds(0, rows_w)],
                        cols_hbm.at[pl.ds(base, rows_w)])
        if extra:
            @pl.when(w >= wcut)
            def _():
                pltpu.sync_copy(cols_v.at[pl.ds(rows_w, L)],
                                cols_hbm.at[pl.ds(base + rows_w, L)])

    return call


def _tc_expand(cols_ref, src_ref, out_ref):
    c = jnp.reshape(cols_ref[...], (BR, 1))
    s = jnp.reshape(src_ref[...], (BR, 1))
    col_iota = lax.broadcasted_iota(jnp.int32, (BR, D), 1)
    out_ref[...] = jnp.where(c == col_iota, s, jnp.float32(0.0))


def kernel(list_scalars, src_vals, one_hot_dim):
    n = list_scalars.shape[0]
    lim = jnp.full((L,), jnp.minimum(one_hot_dim - 1, D - 1), jnp.int32)
    cols = _build_sc_cols(n)(list_scalars, lim)
    grid = n // BR
    cols3 = cols.reshape(grid, 1, BR)
    src3 = src_vals.reshape(grid, 1, BR)
    return pl.pallas_call(
        _tc_expand,
        out_shape=jax.ShapeDtypeStruct((n, D), jnp.float32),
        grid=(grid,),
        in_specs=[
            pl.BlockSpec((1, 1, BR), lambda i: (i, 0, 0)),
            pl.BlockSpec((1, 1, BR), lambda i: (i, 0, 0)),
        ],
        out_specs=pl.BlockSpec((BR, D), lambda i: (i, 0)),
    )(cols3, src3)


# speedup vs baseline: 1.0386x; 1.0386x over previous
import functools

import jax
import jax.numpy as jnp
from jax import lax
from jax.experimental import pallas as pl
from jax.experimental.pallas import tpu as pltpu
from jax.experimental.pallas import tpu_sc as plsc

L = 16
D = 8
NC = 2
NS = 16
NW = NC * NS
BR = 1000


def _build_sc_cols(n):
    groups = n // L
    gbase = groups // NW
    extra = groups - gbase * NW
    wcut = NW - extra
    gmax = gbase + (1 if extra else 0)
    rows_w = gbase * L
    rows_max = gmax * L

    g1 = groups // NS
    rem1 = groups - g1 * NS
    ch1 = g1 * L
    tail_off = ch1 * NS

    mesh = plsc.VectorSubcoreMesh(core_axis_name="c", subcore_axis_name="s")

    @functools.partial(
        pl.kernel,
        out_type=jax.ShapeDtypeStruct((n,), jnp.int32),
        mesh=mesh,
        scratch_types=[
            pltpu.VMEM((ch1,), jnp.int32),
            pltpu.VMEM((L,), jnp.int32),
            pltpu.VMEM((rows_max,), jnp.int32),
            pltpu.VMEM((rows_max,), jnp.int32),
            pltpu.VMEM((L,), jnp.int32),
            pltpu.VMEM((NS * L,), jnp.int32),
            pltpu.VMEM((L,), jnp.int32),
            pltpu.VMEM_SHARED((NS * L,), jnp.int32),
        ],
        compiler_params=pltpu.CompilerParams(needs_layout_passes=False),
    )
    def call(ls_hbm, lim_hbm, cols_hbm,
             vals1_v, tail_v, vals2_v, cols_v,
             minvec_v, allmins_v, lim_v, mins_sh):
        c = lax.axis_index("c")
        s = lax.axis_index("s")
        w = s * NC + c

        pltpu.sync_copy(ls_hbm.at[pl.ds(s * ch1, ch1)], vals1_v)
        if rem1:
            pltpu.sync_copy(
                ls_hbm.at[pl.ds(tail_off + lax.rem(s, rem1) * L, L)], tail_v)
            m0 = tail_v[...]
        else:
            m0 = jnp.full((L,), jnp.iinfo(jnp.int32).max, jnp.int32)

        def mstep(i, m):
            return jnp.minimum(m, vals1_v[pl.ds(i * L, L)])
        m = lax.fori_loop(0, g1, mstep, m0)
        minvec_v[...] = m
        pltpu.sync_copy(minvec_v, mins_sh.at[pl.ds(s * L, L)])
        plsc.subcore_barrier()
        pltpu.sync_copy(mins_sh, allmins_v)

        def mstep2(i, m):
            return jnp.minimum(m, allmins_v[pl.ds(i * L, L)])
        mall = lax.fori_loop(0, NS, mstep2,
                             jnp.full((L,), jnp.iinfo(jnp.int32).max,
                                      jnp.int32))
        gmin = mall[0]
        for j in range(1, L):
            gmin = jnp.minimum(gmin, mall[j])

        base = rows_w * w + L * jnp.maximum(w - wcut, 0)
        ng = gbase + jnp.where(w >= wcut, 1, 0) if extra else gbase
        pltpu.sync_copy(ls_hbm.at[pl.ds(base, rows_max)], vals2_v)
        pltpu.sync_copy(lim_hbm, lim_v)
        lim = lim_v[...]

        def wstep(i, carry):
            v = vals2_v[pl.ds(i * L, L)] - gmin
            v = jnp.minimum(v, lim)
            v = jnp.maximum(v, 0)
            cols_v[pl.ds(i * L, L)] = v
            return carry
        lax.fori_loop(0, ng, wstep, 0)

        pltpu.sync_copy(cols_v.at[pl.ds(0, rows_w)],
                        cols_hbm.at[pl.ds(base, rows_w)])
        if extra:
            @pl.when(w >= wcut)
            def _():
                pltpu.sync_copy(cols_v.at[pl.ds(rows_w, L)],
                                cols_hbm.at[pl.ds(base + rows_w, L)])

    return call


def _tc_expand(cols_ref, src_ref, out_ref):
    c = jnp.reshape(cols_ref[...], (BR,))
    s = jnp.reshape(src_ref[...], (BR,))
    cb = jnp.broadcast_to(c[None, :], (D, BR))
    sb = jnp.broadcast_to(s[None, :], (D, BR))
    row_iota = lax.broadcasted_iota(jnp.int32, (D, BR), 0)
    t = jnp.where(cb == row_iota, sb, jnp.float32(0.0))
    out_ref[...] = t.T


def kernel(list_scalars, src_vals, one_hot_dim):
    n = list_scalars.shape[0]
    lim = jnp.full((L,), jnp.minimum(one_hot_dim - 1, D - 1), jnp.int32)
    cols = _build_sc_cols(n)(list_scalars, lim)
    grid = n // BR
    cols3 = cols.reshape(grid, 1, BR)
    src3 = src_vals.reshape(grid, 1, BR)
    return pl.pallas_call(
        _tc_expand,
        out_shape=jax.ShapeDtypeStruct((n, D), jnp.float32),
        grid=(grid,),
        in_specs=[
            pl.BlockSpec((1, 1, BR), lambda i: (i, 0, 0)),
            pl.BlockSpec((1, 1, BR), lambda i: (i, 0, 0)),
        ],
        out_specs=pl.BlockSpec((BR, D), lambda i: (i, 0)),
    )(cols3, src3)


# speedup vs baseline: 3.7883x; 3.6476x over previous
import functools

import jax
import jax.numpy as jnp
from jax import lax
from jax.experimental import pallas as pl
from jax.experimental.pallas import tpu as pltpu
from jax.experimental.pallas import tpu_sc as plsc

L = 16
D = 8
NC = 2
NS = 16
NW = NC * NS
LANE = 128


def _build_call(n):
    npad = -(-n // LANE) * LANE
    ntiles = npad // LANE
    tbase_w = ntiles // NW
    textra = ntiles - tbase_w * NW
    cols_hi = (tbase_w + 1) * LANE
    cols_lo = tbase_w * LANE
    last_cb = (ntiles - tbase_w) * LANE
    last_full = (n - last_cb) // L * L
    last_tail = (n - last_cb) - last_full
    assert last_tail == 0 and last_full % L == 0

    groups = n // L
    g1 = groups // NS
    rem1 = groups - g1 * NS
    ch1 = g1 * L
    tail_off = ch1 * NS

    mesh = plsc.VectorSubcoreMesh(core_axis_name="c", subcore_axis_name="s")

    @functools.partial(
        pl.kernel,
        out_type=jax.ShapeDtypeStruct((D, npad), jnp.float32),
        mesh=mesh,
        scratch_types=[
            pltpu.VMEM((ch1,), jnp.int32),
            pltpu.VMEM((L,), jnp.int32),
            pltpu.VMEM((cols_hi,), jnp.int32),
            pltpu.VMEM((cols_hi,), jnp.float32),
            pltpu.VMEM((D, cols_hi), jnp.float32),
            pltpu.VMEM((L,), jnp.int32),
            pltpu.VMEM((NS * L,), jnp.int32),
            pltpu.VMEM((L,), jnp.int32),
            pltpu.VMEM_SHARED((NS * L,), jnp.int32),
        ],
        compiler_params=pltpu.CompilerParams(needs_layout_passes=False),
    )
    def call(ls_hbm, sv_hbm, lim_hbm, out_hbm,
             vals1_v, tail_v, vals2_v, src_v, out2_v,
             minvec_v, allmins_v, lim_v, mins_sh):
        c = lax.axis_index("c")
        s = lax.axis_index("s")
        w = s * NC + c

        pltpu.sync_copy(ls_hbm.at[pl.ds(s * ch1, ch1)], vals1_v)
        if rem1:
            pltpu.sync_copy(
                ls_hbm.at[pl.ds(tail_off + lax.rem(s, rem1) * L, L)], tail_v)
            m0 = tail_v[...]
        else:
            m0 = jnp.full((L,), jnp.iinfo(jnp.int32).max, jnp.int32)

        def mstep(i, m):
            return jnp.minimum(m, vals1_v[pl.ds(i * L, L)])
        m = lax.fori_loop(0, g1, mstep, m0)
        minvec_v[...] = m
        pltpu.sync_copy(minvec_v, mins_sh.at[pl.ds(s * L, L)])
        plsc.subcore_barrier()
        pltpu.sync_copy(mins_sh, allmins_v)

        def mstep2(i, m):
            return jnp.minimum(m, allmins_v[pl.ds(i * L, L)])
        mall = lax.fori_loop(0, NS, mstep2,
                             jnp.full((L,), jnp.iinfo(jnp.int32).max,
                                      jnp.int32))
        gmin = mall[0]
        for j in range(1, L):
            gmin = jnp.minimum(gmin, mall[j])

        tb = tbase_w * w + jnp.minimum(w, textra)
        cb = tb * LANE
        hi = w < textra
        last = w >= NW - 1
        ngc = (tbase_w + jnp.where(hi, 1, 0)) * (LANE // L)

        @pl.when(hi)
        def _():
            pltpu.sync_copy(ls_hbm.at[pl.ds(cb, cols_hi)], vals2_v)
            pltpu.sync_copy(sv_hbm.at[pl.ds(cb, cols_hi)], src_v)

        @pl.when(jnp.logical_and(jnp.logical_not(hi),
                                 jnp.logical_not(last)))
        def _():
            pltpu.sync_copy(ls_hbm.at[pl.ds(cb, cols_lo)],
                            vals2_v.at[pl.ds(0, cols_lo)])
            pltpu.sync_copy(sv_hbm.at[pl.ds(cb, cols_lo)],
                            src_v.at[pl.ds(0, cols_lo)])

        @pl.when(last)
        def _():
            pltpu.sync_copy(ls_hbm.at[pl.ds(last_cb, last_full)],
                            vals2_v.at[pl.ds(0, last_full)])
            pltpu.sync_copy(sv_hbm.at[pl.ds(last_cb, last_full)],
                            src_v.at[pl.ds(0, last_full)])

        pltpu.sync_copy(lim_hbm, lim_v)
        lim = lim_v[...]
        fzero = jnp.zeros((L,), jnp.float32)

        def wstep(i, carry):
            v = vals2_v[pl.ds(i * L, L)] - gmin
            v = jnp.minimum(v, lim)
            v = jnp.maximum(v, 0)
            sv = src_v[pl.ds(i * L, L)]
            for d in range(D):
                out2_v[d, pl.ds(i * L, L)] = jnp.where(v == d, sv, fzero)
            return carry
        lax.fori_loop(0, ngc, wstep, 0)

        @pl.when(hi)
        def _():
            pltpu.sync_copy(out2_v, out_hbm.at[:, pl.ds(cb, cols_hi)])

        @pl.when(jnp.logical_not(hi))
        def _():
            pltpu.sync_copy(out2_v.at[:, pl.ds(0, cols_lo)],
                            out_hbm.at[:, pl.ds(cb, cols_lo)])

    return call


def kernel(list_scalars, src_vals, one_hot_dim):
    n = list_scalars.shape[0]
    lim = jnp.full((L,), jnp.minimum(one_hot_dim - 1, D - 1), jnp.int32)
    out_t = _build_call(n)(list_scalars, src_vals, lim)
    return out_t[:, :n].T


# speedup vs baseline: 4.1850x; 1.1047x over previous
import functools

import jax
import jax.numpy as jnp
from jax import lax
from jax.experimental import pallas as pl
from jax.experimental.pallas import tpu as pltpu
from jax.experimental.pallas import tpu_sc as plsc

L = 16
D = 8
NC = 2
NS = 16
NW = NC * NS
LANE = 128


def _build_call(n):
    npad = -(-n // LANE) * LANE
    ntiles = npad // LANE
    tbase_w = ntiles // NW
    textra = ntiles - tbase_w * NW
    cols_hi = (tbase_w + 1) * LANE
    cols_lo = tbase_w * LANE
    shift_max = npad - n + LANE
    cols_buf = cols_hi + shift_max

    groups = n // L
    g1 = groups // NS
    rem1 = groups - g1 * NS
    ch1 = g1 * L
    tail_off = ch1 * NS
    assert g1 % 2 == 0 and tbase_w % 2 == 0 and LANE // L == 8

    mesh = plsc.VectorSubcoreMesh(core_axis_name="c", subcore_axis_name="s")

    @functools.partial(
        pl.kernel,
        out_type=jax.ShapeDtypeStruct((D, npad), jnp.float32),
        mesh=mesh,
        scratch_types=[
            pltpu.VMEM((ch1,), jnp.int32),
            pltpu.VMEM((L,), jnp.int32),
            pltpu.VMEM((cols_buf,), jnp.int32),
            pltpu.VMEM((cols_buf,), jnp.float32),
            pltpu.VMEM((D, cols_hi), jnp.float32),
            pltpu.VMEM((L,), jnp.int32),
            pltpu.VMEM((NS * L,), jnp.int32),
            pltpu.VMEM_SHARED((NS * L,), jnp.int32),
            pltpu.SemaphoreType.DMA,
            pltpu.SemaphoreType.DMA,
        ],
        compiler_params=pltpu.CompilerParams(needs_layout_passes=False),
    )
    def call(ls_hbm, sv_hbm, out_hbm,
             vals1_v, tail_v, vals2_v, src_v, out2_v,
             minvec_v, allmins_v, mins_sh, sem_v, sem_s):
        c = lax.axis_index("c")
        s = lax.axis_index("s")
        w = s * NC + c

        tb = tbase_w * w + jnp.minimum(w, textra)
        cb = tb * LANE
        base2 = jnp.minimum(cb, n - cols_hi)
        shift = cb - base2
        hi = w < textra
        ngc2 = (tbase_w + jnp.where(hi, 1, 0)) * (LANE // L // 2)

        h_v = pltpu.async_copy(ls_hbm.at[pl.ds(base2, cols_hi)],
                               vals2_v.at[pl.ds(0, cols_hi)], sem_v)
        h_s = pltpu.async_copy(sv_hbm.at[pl.ds(base2, cols_hi)],
                               src_v.at[pl.ds(0, cols_hi)], sem_s)

        pltpu.sync_copy(ls_hbm.at[pl.ds(s * ch1, ch1)], vals1_v)
        if rem1:
            pltpu.sync_copy(
                ls_hbm.at[pl.ds(tail_off + lax.rem(s, rem1) * L, L)], tail_v)
            m0 = tail_v[...]
        else:
            m0 = jnp.full((L,), jnp.iinfo(jnp.int32).max, jnp.int32)

        def mstep(i, m):
            m = jnp.minimum(m, vals1_v[pl.ds(i * (2 * L), L)])
            return jnp.minimum(m, vals1_v[pl.ds(i * (2 * L) + L, L)])
        m = lax.fori_loop(0, g1 // 2, mstep, m0)
        minvec_v[...] = m
        pltpu.sync_copy(minvec_v, mins_sh.at[pl.ds(s * L, L)])
        plsc.subcore_barrier()
        pltpu.sync_copy(mins_sh, allmins_v)

        def mstep2(i, m):
            return jnp.minimum(m, allmins_v[pl.ds(i * L, L)])
        mall = lax.fori_loop(0, NS, mstep2,
                             jnp.full((L,), jnp.iinfo(jnp.int32).max,
                                      jnp.int32))
        gmin = mall[0]
        for j in range(1, L):
            gmin = jnp.minimum(gmin, mall[j])

        h_v.wait()
        h_s.wait()
        fzero = jnp.zeros((L,), jnp.float32)

        def group(g):
            o = shift + g * L
            v = vals2_v[pl.ds(o, L)] - gmin
            v = jnp.minimum(v, D - 1)
            v = jnp.maximum(v, 0)
            sv = src_v[pl.ds(o, L)]
            for d in range(D):
                out2_v[d, pl.ds(g * L, L)] = jnp.where(v == d, sv, fzero)

        def wstep(i, carry):
            group(2 * i)
            group(2 * i + 1)
            return carry
        lax.fori_loop(0, ngc2, wstep, 0)

        @pl.when(hi)
        def _():
            pltpu.sync_copy(out2_v, out_hbm.at[:, pl.ds(cb, cols_hi)])

        @pl.when(jnp.logical_not(hi))
        def _():
            pltpu.sync_copy(out2_v.at[:, pl.ds(0, cols_lo)],
                            out_hbm.at[:, pl.ds(cb, cols_lo)])

    return call


def kernel(list_scalars, src_vals, one_hot_dim):
    del one_hot_dim
    n = list_scalars.shape[0]
    out_t = _build_call(n)(list_scalars, src_vals)
    return out_t[:, :n].T


# speedup vs baseline: 4.2418x; 1.0136x over previous
import functools

import jax
import jax.numpy as jnp
from jax import lax
from jax.experimental import pallas as pl
from jax.experimental.pallas import tpu as pltpu
from jax.experimental.pallas import tpu_sc as plsc

L = 16
D = 8
NC = 2
NS = 16
NW = NC * NS
LANE = 128


def _build_call(n):
    npad = -(-n // LANE) * LANE
    ntiles = npad // LANE
    tbase_w = ntiles // NW
    textra = ntiles - tbase_w * NW
    cols_hi = (tbase_w + 1) * LANE
    cols_lo = tbase_w * LANE
    shift_max = npad - n + LANE
    cols_buf = cols_hi + shift_max

    groups = n // L
    g1 = groups // NS
    rem1 = groups - g1 * NS
    ch1 = g1 * L
    tail_off = ch1 * NS
    assert g1 % 2 == 0 and tbase_w % 2 == 0 and LANE // L == 8

    mesh = plsc.VectorSubcoreMesh(core_axis_name="c", subcore_axis_name="s")

    @functools.partial(
        pl.kernel,
        out_type=jax.ShapeDtypeStruct((D, npad), jnp.float32),
        mesh=mesh,
        scratch_types=[
            pltpu.VMEM((ch1,), jnp.int32),
            pltpu.VMEM((L,), jnp.int32),
            pltpu.VMEM((cols_buf,), jnp.int32),
            pltpu.VMEM((cols_buf,), jnp.float32),
            pltpu.VMEM((D, cols_hi), jnp.float32),
            pltpu.VMEM((L,), jnp.int32),
            pltpu.VMEM((NS * L,), jnp.int32),
            pltpu.VMEM_SHARED((NS * L,), jnp.int32),
            pltpu.SemaphoreType.DMA,
            pltpu.SemaphoreType.DMA,
            pltpu.SemaphoreType.DMA,
        ],
        compiler_params=pltpu.CompilerParams(needs_layout_passes=False),
    )
    def call(ls_hbm, sv_hbm, out_hbm,
             vals1_v, tail_v, vals2_v, src_v, out2_v,
             minvec_v, allmins_v, mins_sh, sem_v, sem_s, sem_o):
        c = lax.axis_index("c")
        s = lax.axis_index("s")
        w = s * NC + c

        tb = tbase_w * w + jnp.minimum(w, textra)
        cb = tb * LANE
        base2 = jnp.minimum(cb, n - cols_hi)
        shift = cb - base2
        hi = w < textra
        ngc2 = (tbase_w + jnp.where(hi, 1, 0)) * (LANE // L // 2)

        h_v = pltpu.async_copy(ls_hbm.at[pl.ds(base2, cols_hi)],
                               vals2_v.at[pl.ds(0, cols_hi)], sem_v)
        h_s = pltpu.async_copy(sv_hbm.at[pl.ds(base2, cols_hi)],
                               src_v.at[pl.ds(0, cols_hi)], sem_s)

        pltpu.sync_copy(ls_hbm.at[pl.ds(s * ch1, ch1)], vals1_v)
        if rem1:
            pltpu.sync_copy(
                ls_hbm.at[pl.ds(tail_off + lax.rem(s, rem1) * L, L)], tail_v)
            m0 = tail_v[...]
        else:
            m0 = jnp.full((L,), jnp.iinfo(jnp.int32).max, jnp.int32)

        def mstep(i, m):
            m = jnp.minimum(m, vals1_v[pl.ds(i * (2 * L), L)])
            return jnp.minimum(m, vals1_v[pl.ds(i * (2 * L) + L, L)])
        m = lax.fori_loop(0, g1 // 2, mstep, m0)
        minvec_v[...] = m
        pltpu.sync_copy(minvec_v, mins_sh.at[pl.ds(s * L, L)])
        plsc.subcore_barrier()
        pltpu.sync_copy(mins_sh, allmins_v)

        def mstep2(i, m):
            return jnp.minimum(m, allmins_v[pl.ds(i * L, L)])
        mall = lax.fori_loop(0, NS, mstep2,
                             jnp.full((L,), jnp.iinfo(jnp.int32).max,
                                      jnp.int32))
        gmin = mall[0]
        for j in range(1, L):
            gmin = jnp.minimum(gmin, mall[j])

        h_v.wait()
        h_s.wait()
        fzero = jnp.zeros((L,), jnp.float32)
        lane = lax.iota(jnp.int32, L)

        def group(g):
            o = shift + g * L
            v = vals2_v[pl.ds(o, L)] - gmin
            v = jnp.minimum(v, D - 1)
            v = jnp.maximum(v, 0)
            sv = src_v[pl.ds(o, L)]
            for d in range(D):
                out2_v[d, pl.ds(g * L, L)] = fzero
            plsc.store_scatter(out2_v, [v, g * L + lane], sv)

        def wstep(i, carry):
            group(2 * i)
            group(2 * i + 1)
            return carry

        ghalf = cols_lo // L // 2
        lax.fori_loop(0, ghalf // 2, wstep, 0)
        h_o = pltpu.async_copy(out2_v.at[:, pl.ds(0, cols_lo // 2)],
                               out_hbm.at[:, pl.ds(cb, cols_lo // 2)],
                               sem_o)
        lax.fori_loop(ghalf // 2, ngc2, wstep, 0)

        @pl.when(hi)
        def _():
            pltpu.sync_copy(
                out2_v.at[:, pl.ds(cols_lo // 2, cols_hi - cols_lo // 2)],
                out_hbm.at[:, pl.ds(cb + cols_lo // 2,
                                    cols_hi - cols_lo // 2)])

        @pl.when(jnp.logical_not(hi))
        def _():
            pltpu.sync_copy(
                out2_v.at[:, pl.ds(cols_lo // 2, cols_lo // 2)],
                out_hbm.at[:, pl.ds(cb + cols_lo // 2, cols_lo // 2)])
        h_o.wait()

    return call


def kernel(list_scalars, src_vals, one_hot_dim):
    del one_hot_dim
    n = list_scalars.shape[0]
    out_t = _build_call(n)(list_scalars, src_vals)
    return out_t[:, :n].T


# speedup vs baseline: 4.8965x; 1.1543x over previous
import functools

import jax
import jax.numpy as jnp
from jax import lax
from jax.experimental import pallas as pl
from jax.experimental.pallas import tpu as pltpu
from jax.experimental.pallas import tpu_sc as plsc

L = 16
D = 8
NC = 2
NS = 16
NW = NC * NS
LANE = 128


def _build_call(n):
    npad = -(-n // LANE) * LANE
    ntiles = npad // LANE
    tbase_w = ntiles // NW
    textra = ntiles - tbase_w * NW
    cols_hi = (tbase_w + 1) * LANE
    cols_lo = tbase_w * LANE
    shift_max = npad - n + LANE
    cols_buf = cols_hi + shift_max

    groups = n // L
    g1 = groups // NS
    rem1 = groups - g1 * NS
    ch1 = g1 * L
    tail_off = ch1 * NS
    assert g1 % 2 == 0 and tbase_w % 2 == 0 and LANE // L == 8

    mesh = plsc.VectorSubcoreMesh(core_axis_name="c", subcore_axis_name="s")

    @functools.partial(
        pl.kernel,
        out_type=jax.ShapeDtypeStruct((D, n), jnp.float32),
        mesh=mesh,
        scratch_types=[
            pltpu.VMEM((ch1,), jnp.int32),
            pltpu.VMEM((L,), jnp.int32),
            pltpu.VMEM((cols_buf,), jnp.int32),
            pltpu.VMEM((cols_buf,), jnp.float32),
            pltpu.VMEM((D, cols_hi), jnp.float32),
            pltpu.VMEM((L,), jnp.int32),
            pltpu.VMEM((NS * L,), jnp.int32),
            pltpu.VMEM_SHARED((NS * L,), jnp.int32),
            pltpu.SemaphoreType.DMA,
            pltpu.SemaphoreType.DMA,
            pltpu.SemaphoreType.DMA,
        ],
        compiler_params=pltpu.CompilerParams(needs_layout_passes=False),
    )
    def call(ls_hbm, sv_hbm, out_hbm,
             vals1_v, tail_v, vals2_v, src_v, out2_v,
             minvec_v, allmins_v, mins_sh, sem_v, sem_s, sem_o):
        c = lax.axis_index("c")
        s = lax.axis_index("s")
        w = s * NC + c

        tb = tbase_w * w + jnp.minimum(w, textra)
        cb = tb * LANE
        base2 = jnp.minimum(cb, n - cols_hi)
        shift = cb - base2
        hi = w < textra
        ngc2 = (tbase_w + jnp.where(hi, 1, 0)) * (LANE // L // 2)

        h_v = pltpu.async_copy(ls_hbm.at[pl.ds(base2, cols_hi)],
                               vals2_v.at[pl.ds(0, cols_hi)], sem_v)
        h_s = pltpu.async_copy(sv_hbm.at[pl.ds(base2, cols_hi)],
                               src_v.at[pl.ds(0, cols_hi)], sem_s)

        pltpu.sync_copy(ls_hbm.at[pl.ds(s * ch1, ch1)], vals1_v)
        if rem1:
            pltpu.sync_copy(
                ls_hbm.at[pl.ds(tail_off + lax.rem(s, rem1) * L, L)], tail_v)
            m0 = tail_v[...]
        else:
            m0 = jnp.full((L,), jnp.iinfo(jnp.int32).max, jnp.int32)

        def mstep(i, m):
            m = jnp.minimum(m, vals1_v[pl.ds(i * (2 * L), L)])
            return jnp.minimum(m, vals1_v[pl.ds(i * (2 * L) + L, L)])
        m = lax.fori_loop(0, g1 // 2, mstep, m0)
        minvec_v[...] = m
        pltpu.sync_copy(minvec_v, mins_sh.at[pl.ds(s * L, L)])
        plsc.subcore_barrier()
        pltpu.sync_copy(mins_sh, allmins_v)

        def mstep2(i, m):
            return jnp.minimum(m, allmins_v[pl.ds(i * L, L)])
        mall = lax.fori_loop(0, NS, mstep2,
                             jnp.full((L,), jnp.iinfo(jnp.int32).max,
                                      jnp.int32))
        gmin = mall[0]
        for j in range(1, L):
            gmin = jnp.minimum(gmin, mall[j])

        h_v.wait()
        h_s.wait()
        fzero = jnp.zeros((L,), jnp.float32)
        lane = lax.iota(jnp.int32, L)

        def group(g):
            o = shift + g * L
            v = vals2_v[pl.ds(o, L)] - gmin
            v = jnp.minimum(v, D - 1)
            v = jnp.maximum(v, 0)
            sv = src_v[pl.ds(o, L)]
            for d in range(D):
                out2_v[d, pl.ds(g * L, L)] = fzero
            plsc.store_scatter(out2_v, [v, g * L + lane], sv)

        def wstep(i, carry):
            group(2 * i)
            group(2 * i + 1)
            return carry

        ghalf = cols_lo // L // 2
        lax.fori_loop(0, ghalf // 2, wstep, 0)
        h_o = pltpu.async_copy(out2_v.at[:, pl.ds(0, cols_lo // 2)],
                               out_hbm.at[:, pl.ds(cb, cols_lo // 2)],
                               sem_o)
        lax.fori_loop(ghalf // 2, ngc2, wstep, 0)

        last = w >= NW - 1
        half = cols_lo // 2
        cols_last = n - (ntiles - tbase_w) * LANE

        @pl.when(hi)
        def _():
            pltpu.sync_copy(
                out2_v.at[:, pl.ds(half, cols_hi - half)],
                out_hbm.at[:, pl.ds(cb + half, cols_hi - half)])

        @pl.when(jnp.logical_and(jnp.logical_not(hi),
                                 jnp.logical_not(last)))
        def _():
            pltpu.sync_copy(
                out2_v.at[:, pl.ds(half, cols_lo - half)],
                out_hbm.at[:, pl.ds(cb + half, cols_lo - half)])

        cols_full = cols_last // LANE * LANE
        cols_edge = cols_last - cols_full

        @pl.when(last)
        def _():
            pltpu.sync_copy(
                out2_v.at[:, pl.ds(half, cols_full - half)],
                out_hbm.at[:, pl.ds(cb + half, cols_full - half)])
            pltpu.sync_copy(
                out2_v.at[:, pl.ds(cols_full, cols_edge)],
                out_hbm.at[:, pl.ds(cb + cols_full, cols_edge)])

        h_o.wait()

    return call


def kernel(list_scalars, src_vals, one_hot_dim):
    del one_hot_dim
    n = list_scalars.shape[0]
    out_t = _build_call(n)(list_scalars, src_vals)
    return out_t.T


# speedup vs baseline: 5.0134x; 1.0239x over previous
import functools

import jax
import jax.numpy as jnp
from jax import lax
from jax.experimental import pallas as pl
from jax.experimental.pallas import tpu as pltpu
from jax.experimental.pallas import tpu_sc as plsc

L = 16
D = 8
NC = 2
NS = 16
NW = NC * NS
LANE = 128


def _build_call(n):
    npad = -(-n // LANE) * LANE
    ntiles = npad // LANE
    tbase_w = ntiles // NW
    textra = ntiles - tbase_w * NW
    cols_hi = (tbase_w + 1) * LANE
    cols_lo = tbase_w * LANE
    shift_max = npad - n + LANE
    cols_buf = cols_hi + shift_max

    win = 2 * cols_hi
    win_buf = win + (npad - n)
    assert tbase_w % 2 == 0 and LANE // L == 8 and win // L % 2 == 0

    mesh = plsc.VectorSubcoreMesh(core_axis_name="c", subcore_axis_name="s")

    @functools.partial(
        pl.kernel,
        out_type=jax.ShapeDtypeStruct((D, n), jnp.float32),
        mesh=mesh,
        scratch_types=[
            pltpu.VMEM((win_buf,), jnp.int32),
            pltpu.VMEM((cols_buf,), jnp.float32),
            pltpu.VMEM((D, cols_hi), jnp.float32),
            pltpu.VMEM((L,), jnp.int32),
            pltpu.VMEM((NS * L,), jnp.int32),
            pltpu.VMEM_SHARED((NS * L,), jnp.int32),
            pltpu.SemaphoreType.DMA,
            pltpu.SemaphoreType.DMA,
            pltpu.SemaphoreType.DMA,
        ],
        compiler_params=pltpu.CompilerParams(needs_layout_passes=False),
    )
    def call(ls_hbm, sv_hbm, out_hbm,
             vals2_v, src_v, out2_v,
             minvec_v, allmins_v, mins_sh, sem_v, sem_s, sem_o):
        c = lax.axis_index("c")
        s = lax.axis_index("s")
        w = s * NC + c

        tb = tbase_w * w + jnp.minimum(w, textra)
        cb = tb * LANE
        base_s = jnp.minimum((tbase_w * (2 * s)
                              + jnp.minimum(2 * s, textra)) * LANE, n - win)
        shift2 = cb - base_s
        base2 = jnp.minimum(cb, n - cols_hi)
        shift = cb - base2
        hi = w < textra
        ngc2 = (tbase_w + jnp.where(hi, 1, 0)) * (LANE // L // 2)

        h_v = pltpu.async_copy(ls_hbm.at[pl.ds(base_s, win)],
                               vals2_v.at[pl.ds(0, win)], sem_v)
        h_s = pltpu.async_copy(sv_hbm.at[pl.ds(base2, cols_hi)],
                               src_v.at[pl.ds(0, cols_hi)], sem_s)

        h_v.wait()

        def mstep(i, m):
            m = jnp.minimum(m, vals2_v[pl.ds(i * (2 * L), L)])
            return jnp.minimum(m, vals2_v[pl.ds(i * (2 * L) + L, L)])
        m = lax.fori_loop(0, win // L // 2, mstep,
                          jnp.full((L,), jnp.iinfo(jnp.int32).max,
                                   jnp.int32))
        minvec_v[...] = m
        pltpu.sync_copy(minvec_v, mins_sh.at[pl.ds(s * L, L)])
        plsc.subcore_barrier()
        pltpu.sync_copy(mins_sh, allmins_v)

        def mstep2(i, m):
            return jnp.minimum(m, allmins_v[pl.ds(i * L, L)])
        mall = lax.fori_loop(0, NS, mstep2,
                             jnp.full((L,), jnp.iinfo(jnp.int32).max,
                                      jnp.int32))
        gmin = mall[0]
        for j in range(1, L):
            gmin = jnp.minimum(gmin, mall[j])

        h_s.wait()
        fzero = jnp.zeros((L,), jnp.float32)
        lane = lax.iota(jnp.int32, L)

        def group(g):
            v = vals2_v[pl.ds(shift2 + g * L, L)] - gmin
            v = jnp.minimum(v, D - 1)
            v = jnp.maximum(v, 0)
            sv = src_v[pl.ds(shift + g * L, L)]
            for d in range(D):
                out2_v[d, pl.ds(g * L, L)] = fzero
            plsc.store_scatter(out2_v, [v, g * L + lane], sv)

        def wstep(i, carry):
            group(2 * i)
            group(2 * i + 1)
            return carry

        ghalf = cols_lo // L // 2
        lax.fori_loop(0, ghalf // 2, wstep, 0)
        h_o = pltpu.async_copy(out2_v.at[:, pl.ds(0, cols_lo // 2)],
                               out_hbm.at[:, pl.ds(cb, cols_lo // 2)],
                               sem_o)
        lax.fori_loop(ghalf // 2, ngc2, wstep, 0)

        last = w >= NW - 1
        half = cols_lo // 2
        cols_last = n - (ntiles - tbase_w) * LANE

        @pl.when(hi)
        def _():
            pltpu.sync_copy(
                out2_v.at[:, pl.ds(half, cols_hi - half)],
                out_hbm.at[:, pl.ds(cb + half, cols_hi - half)])

        @pl.when(jnp.logical_and(jnp.logical_not(hi),
                                 jnp.logical_not(last)))
        def _():
            pltpu.sync_copy(
                out2_v.at[:, pl.ds(half, cols_lo - half)],
                out_hbm.at[:, pl.ds(cb + half, cols_lo - half)])

        cols_full = cols_last // LANE * LANE
        cols_edge = cols_last - cols_full

        @pl.when(last)
        def _():
            pltpu.sync_copy(
                out2_v.at[:, pl.ds(half, cols_full - half)],
                out_hbm.at[:, pl.ds(cb + half, cols_full - half)])
            pltpu.sync_copy(
                out2_v.at[:, pl.ds(cols_full, cols_edge)],
                out_hbm.at[:, pl.ds(cb + cols_full, cols_edge)])

        h_o.wait()

    return call


def kernel(list_scalars, src_vals, one_hot_dim):
    del one_hot_dim
    n = list_scalars.shape[0]
    out_t = _build_call(n)(list_scalars, src_vals)
    return out_t.T
